# baseline (speedup 1.0000x reference)
import functools

import jax
import jax.numpy as jnp
from jax import lax
from jax.experimental import pallas as pl
from jax.experimental.pallas import tpu as pltpu
from jax.experimental.pallas import tpu_sc as plsc

NUM_FIELDS = 26
EMBEDDING_DIM = 32
LANES = 16
ROW_W = 128
DMAS_PER_CHUNK = 8


def _make_sc_gather(n_flat, dim):
    info = plsc.get_sparse_core_info()
    nc, ns = info.num_cores, info.num_subcores
    nw = nc * ns
    per_w = n_flat // nw
    assert per_w * nw == n_flat and per_w % (ROW_W * DMAS_PER_CHUNK) == 0
    n_idx_rows = per_w // ROW_W
    chunk = ROW_W * DMAS_PER_CHUNK
    n_chunks = per_w // chunk

    mesh = plsc.VectorSubcoreMesh(core_axis_name="c", subcore_axis_name="s")

    @functools.partial(
        pl.kernel,
        mesh=mesh,
        compiler_params=pltpu.CompilerParams(use_tc_tiling_on_sc=False),
        out_type=jax.ShapeDtypeStruct((n_flat, dim), jnp.float32),
        scratch_types=[
            pltpu.VMEM((n_idx_rows, ROW_W), jnp.int32),
            pltpu.VMEM((n_idx_rows, ROW_W), jnp.int32),
            pltpu.VMEM((chunk, dim), jnp.float32),
            pltpu.SemaphoreType.DMA,
        ],
    )
    def sc_gather(idx_hbm, off_hbm, table_hbm, out_hbm, idx_v, off_v, rows_v, gsem):
        wid = lax.axis_index("s") * nc + lax.axis_index("c")
        base_row = wid * n_idx_rows
        pltpu.sync_copy(idx_hbm.at[pl.ds(base_row, n_idx_rows)], idx_v)
        pltpu.sync_copy(off_hbm, off_v)

        def add_body(r, carry):
            for k in range(ROW_W // LANES):
                sl = pl.ds(k * LANES, LANES)
                idx_v[r, sl] = idx_v[r, sl] + off_v[r, sl]
            return carry

        lax.fori_loop(0, n_idx_rows, add_body, 0)

        def chunk_body(t, carry):
            copies = []
            for b in range(DMAS_PER_CHUNK):
                copies.append(pltpu.async_copy(
                    table_hbm.at[idx_v.at[t * DMAS_PER_CHUNK + b]],
                    rows_v.at[pl.ds(b * ROW_W, ROW_W)],
                    gsem,
                ))
            for c in copies:
                c.wait()
            pltpu.sync_copy(
                rows_v,
                out_hbm.at[pl.ds((base_row + t * DMAS_PER_CHUNK) * ROW_W, chunk)],
            )
            return carry

        lax.fori_loop(0, n_chunks, chunk_body, 0)

    return sc_gather


def kernel(categorical_inputs, embedding_weight, offsets):
    batch, n_fields = categorical_inputs.shape
    n_flat = batch * n_fields
    idx_flat = categorical_inputs.astype(jnp.int32).reshape(n_flat // ROW_W, ROW_W)
    info = plsc.get_sparse_core_info()
    per_w = n_flat // (info.num_cores * info.num_subcores)
    off_pattern = jnp.tile(
        offsets[:n_fields].astype(jnp.int32), per_w // n_fields
    ).reshape(per_w // ROW_W, ROW_W)
    sc_gather = _make_sc_gather(n_flat, EMBEDDING_DIM)
    out_flat = sc_gather(idx_flat, off_pattern, embedding_weight)
    return out_flat.reshape(batch, n_fields, EMBEDDING_DIM)
